# baseline (speedup 1.0000x reference)
import jax
import jax.numpy as jnp
from jax.experimental import pallas as pl

NUM_TOKENS = 50
EMBED_DIM = 4096


def _noop_body(x_ref, o_ref):
    o_ref[...] = jnp.zeros_like(o_ref)


def kernel(embeds, role_id):
    del role_id
    return pl.pallas_call(
        _noop_body,
        in_specs=[pl.BlockSpec(memory_space=pl.ANY)],
        out_shape=jax.ShapeDtypeStruct((NUM_TOKENS, EMBED_DIM), jnp.float32),
    )(embeds)


# speedup vs baseline: 9.7404x; 9.7404x over previous
import jax
import jax.numpy as jnp
from jax.experimental import pallas as pl
from jax.experimental.pallas import tpu as pltpu

NUM_ROLES = 100
NUM_TOKENS = 50
EMBED_DIM = 4096
BT = 8
BR = 8


def _select_body(rid_ref, x_ref, o_ref):
    r8 = rid_ref[0] % BR
    roles = jax.lax.broadcasted_iota(jnp.int32, (BT, BR, EMBED_DIM), 1)
    onehot = jnp.where(roles == r8, 1.0, 0.0).astype(jnp.float32)
    o_ref[...] = jnp.sum(x_ref[...] * onehot, axis=1)


def kernel(embeds, role_id):
    x = jnp.transpose(embeds, (1, 0, 2))
    rid = jnp.asarray(role_id, jnp.int32).reshape(1)
    grid = (NUM_TOKENS + BT - 1) // BT
    return pl.pallas_call(
        _select_body,
        grid_spec=pltpu.PrefetchScalarGridSpec(
            num_scalar_prefetch=1,
            grid=(grid,),
            in_specs=[
                pl.BlockSpec(
                    (BT, BR, EMBED_DIM),
                    lambda i, rid_ref: (i, rid_ref[0] // BR, 0),
                ),
            ],
            out_specs=pl.BlockSpec((BT, EMBED_DIM), lambda i, rid_ref: (i, 0)),
        ),
        out_shape=jax.ShapeDtypeStruct((NUM_TOKENS, EMBED_DIM), jnp.float32),
    )(rid, x)


# speedup vs baseline: 10.9786x; 1.1271x over previous
import jax
import jax.numpy as jnp
from jax.experimental import pallas as pl
from jax.experimental.pallas import tpu as pltpu

NUM_ROLES = 100
NUM_TOKENS = 50
EMBED_DIM = 4096
BT = 8
BR = 8


def _select_body(rid_ref, x_ref, o_ref):
    r8 = rid_ref[0] % BR
    o_ref[...] = x_ref[:, r8, :]


def kernel(embeds, role_id):
    x = jnp.transpose(embeds, (1, 0, 2))
    rid = jnp.asarray(role_id, jnp.int32).reshape(1)
    grid = (NUM_TOKENS + BT - 1) // BT
    return pl.pallas_call(
        _select_body,
        grid_spec=pltpu.PrefetchScalarGridSpec(
            num_scalar_prefetch=1,
            grid=(grid,),
            in_specs=[
                pl.BlockSpec(
                    (BT, BR, EMBED_DIM),
                    lambda i, rid_ref: (i, rid_ref[0] // BR, 0),
                ),
            ],
            out_specs=pl.BlockSpec((BT, EMBED_DIM), lambda i, rid_ref: (i, 0)),
        ),
        out_shape=jax.ShapeDtypeStruct((NUM_TOKENS, EMBED_DIM), jnp.float32),
    )(rid, x)


# speedup vs baseline: 13.8856x; 1.2648x over previous
import jax
import jax.numpy as jnp
from jax.experimental import pallas as pl
from jax.experimental.pallas import tpu as pltpu

NUM_ROLES = 100
NUM_TOKENS = 50
EMBED_DIM = 4096
BT = 16
BR = 8


def _select_body(rid_ref, x_ref, o_ref):
    r8 = rid_ref[0] % BR
    o_ref[...] = x_ref[:, r8, :]


def kernel(embeds, role_id):
    x = jnp.transpose(embeds, (1, 0, 2))
    rid = jnp.asarray(role_id, jnp.int32).reshape(1)
    grid = (NUM_TOKENS + BT - 1) // BT
    return pl.pallas_call(
        _select_body,
        grid_spec=pltpu.PrefetchScalarGridSpec(
            num_scalar_prefetch=1,
            grid=(grid,),
            in_specs=[
                pl.BlockSpec(
                    (BT, BR, EMBED_DIM),
                    lambda i, rid_ref: (i, rid_ref[0] // BR, 0),
                ),
            ],
            out_specs=pl.BlockSpec((BT, EMBED_DIM), lambda i, rid_ref: (i, 0)),
        ),
        out_shape=jax.ShapeDtypeStruct((NUM_TOKENS, EMBED_DIM), jnp.float32),
    )(rid, x)


# speedup vs baseline: 16.1627x; 1.1640x over previous
import jax
import jax.numpy as jnp
from jax.experimental import pallas as pl
from jax.experimental.pallas import tpu as pltpu

NUM_ROLES = 100
NUM_TOKENS = 50
EMBED_DIM = 4096
BT = 32
BR = 8


def _select_body(rid_ref, x_ref, o_ref):
    r8 = rid_ref[0] % BR
    o_ref[...] = x_ref[:, r8, :]


def kernel(embeds, role_id):
    x = jnp.transpose(embeds, (1, 0, 2))
    rid = jnp.asarray(role_id, jnp.int32).reshape(1)
    grid = (NUM_TOKENS + BT - 1) // BT
    return pl.pallas_call(
        _select_body,
        grid_spec=pltpu.PrefetchScalarGridSpec(
            num_scalar_prefetch=1,
            grid=(grid,),
            in_specs=[
                pl.BlockSpec(
                    (BT, BR, EMBED_DIM),
                    lambda i, rid_ref: (i, rid_ref[0] // BR, 0),
                ),
            ],
            out_specs=pl.BlockSpec((BT, EMBED_DIM), lambda i, rid_ref: (i, 0)),
        ),
        out_shape=jax.ShapeDtypeStruct((NUM_TOKENS, EMBED_DIM), jnp.float32),
    )(rid, x)
